# Initial kernel scaffold; baseline (speedup 1.0000x reference)
import jax
import jax.numpy as jnp
from jax.experimental import pallas as pl


def kernel(hint_scalar, hint_pointer, current_hidden, edge_index, step, W_scalar, b_scalar, W_ptr, b_ptr):
    raise NotImplementedError("write your pallas kernel here")


# speedup vs baseline: 4.4268x; 4.4268x over previous
import functools

import jax
import jax.numpy as jnp
from jax import lax
from jax.experimental import pallas as pl
from jax.experimental.pallas import tpu as pltpu
from jax.experimental.pallas import tpu_sc as plsc

N = 10000
E = 320000
H = 128

NC = 2
NS = 16
NW = NC * NS

EPW = E // NW
C = 80
NCHUNK = EPW // C
N_PAD = 10240
RPT = N_PAD // NS


def _sc_aggregate(src_hbm, dst_hbm, prob_hbm, hidden_hbm, out_hbm,
                  src_v, dst_v, prob_v, rows_v, acc, sem):
    c = lax.axis_index("c")
    s = lax.axis_index("s")
    wid = c * NS + s

    def _zero_row_blk(i, _):
        for r in range(8):
            for g in range(H // 16):
                rows_v[i * 8 + r, pl.ds(g * 16, 16)] = jnp.zeros((16,), jnp.float32)
        return 0
    lax.fori_loop(0, C // 8, _zero_row_blk, 0)
    base_row = s * RPT
    for j in range(RPT // C):
        pltpu.sync_copy(rows_v, acc.at[pl.ds(base_row + j * C, C)])
    plsc.subcore_barrier()

    def _chunk(k, _):
        base = wid * EPW + k * C
        pltpu.sync_copy(dst_hbm.at[pl.ds(base, C)], dst_v)
        pltpu.sync_copy(src_hbm.at[pl.ds(base, C)], src_v)
        pltpu.sync_copy(prob_hbm.at[pl.ds(base, C)], prob_v)
        pltpu.async_copy(hidden_hbm.at[dst_v], rows_v, sem).wait()

        def _scale_blk(i, _):
            pv = prob_v[pl.ds(i * 16, 16)]
            for r in range(16):
                row = i * 16 + r
                p = pv[r]
                for g in range(H // 16):
                    sl = pl.ds(g * 16, 16)
                    rows_v[row, sl] = rows_v[row, sl] * p
            return 0
        lax.fori_loop(0, C // 16, _scale_blk, 0)

        pltpu.sync_copy(rows_v, acc.at[src_v], add=True)
        return 0
    lax.fori_loop(0, NCHUNK, _chunk, 0)
    plsc.subcore_barrier()

    for j in range(RPT // C):
        r0 = base_row + j * C
        pltpu.sync_copy(acc.at[pl.ds(r0, C)], rows_v)
        pltpu.sync_copy(rows_v, out_hbm.at[c, pl.ds(r0, C)])


_sc_call = functools.partial(
    pl.kernel,
    out_type=jax.ShapeDtypeStruct((NC, N_PAD, H), jnp.float32),
    mesh=plsc.VectorSubcoreMesh(core_axis_name="c", subcore_axis_name="s"),
    scratch_types=[
        pltpu.VMEM((C,), jnp.int32),
        pltpu.VMEM((C,), jnp.int32),
        pltpu.VMEM((C,), jnp.float32),
        pltpu.VMEM((C, H), jnp.float32),
        pltpu.VMEM_SHARED((N_PAD, H), jnp.float32),
        pltpu.SemaphoreType.DMA,
    ],
)(_sc_aggregate)


def _tc_project(partials_ref, hs_ref, wrow_ref, wptr_ref, bias_ref, out_ref):
    agg = partials_ref[0, pl.ds(0, N)] + partials_ref[1, pl.ds(0, N)]
    enc_ptr = lax.dot_general(agg, wptr_ref[...], (((1,), (1,)), ((), ())),
                              preferred_element_type=jnp.float32)
    out_ref[...] = enc_ptr + hs_ref[...] * wrow_ref[...] + bias_ref[...]


def kernel(hint_scalar, hint_pointer, current_hidden, edge_index, step,
           W_scalar, b_scalar, W_ptr, b_ptr):
    src = edge_index[0]
    dst = edge_index[1]
    prob = jnp.take(hint_pointer, step, axis=1)
    hs = jnp.take(hint_scalar, step, axis=1)[:, None]

    partials = _sc_call(src, dst, prob, current_hidden)

    wrow = W_scalar.reshape(1, H)
    bias = (b_scalar + b_ptr).reshape(1, H)
    out = pl.pallas_call(
        _tc_project,
        out_shape=jax.ShapeDtypeStruct((N, H), jnp.float32),
    )(partials, hs, wrow, W_ptr, bias)
    return out


# speedup vs baseline: 11.9518x; 2.6999x over previous
import functools

import jax
import jax.numpy as jnp
from jax import lax
from jax.experimental import pallas as pl
from jax.experimental.pallas import tpu as pltpu
from jax.experimental.pallas import tpu_sc as plsc

N = 10000
E = 320000
H = 128

NC = 2
NS = 16
NW = NC * NS

EPW = E // NW
C = 80
NCHUNK = EPW // C
N_PAD = 10112
RPT = N_PAD // NS
NBUF = 4


def _sc_aggregate(packed_hbm, prob_hbm, hidden_hbm, out_hbm,
                  pk0, pk1, pk2, pk3, src0, src1, src2, src3,
                  dst0, dst1, dst2, dst3, pr0, pr1, pr2, pr3,
                  rows0, rows1, rows2, rows3, acc,
                  sem_pk, sem_pr, sem_g, sem_s):
    c = lax.axis_index("c")
    s = lax.axis_index("s")
    wid = c * NS + s
    pk = (pk0, pk1, pk2, pk3)
    src = (src0, src1, src2, src3)
    dst = (dst0, dst1, dst2, dst3)
    pr = (pr0, pr1, pr2, pr3)
    rows = (rows0, rows1, rows2, rows3)

    ebase = wid * EPW

    def issue_packed(b, k):
        pltpu.async_copy(packed_hbm.at[pl.ds(ebase + k * C, C)], pk[b], sem_pk[b])

    def wait_packed(b):
        pltpu.make_async_copy(packed_hbm.at[pl.ds(0, C)], pk[b], sem_pk[b]).wait()

    def issue_prob(b, k):
        pltpu.async_copy(prob_hbm.at[pl.ds(ebase + k * C, C)], pr[b], sem_pr[b])

    def wait_prob(b):
        pltpu.make_async_copy(prob_hbm.at[pl.ds(0, C)], pr[b], sem_pr[b]).wait()

    def unpack(b):
        for i in range(C // 16):
            sl = pl.ds(i * 16, 16)
            v = pk[b][sl]
            dst[b][sl] = jnp.bitwise_and(v, 16383)
            src[b][sl] = jnp.right_shift(v, 14)

    def issue_gather(b):
        pltpu.async_copy(hidden_hbm.at[dst[b]], rows[b], sem_g[b])

    def wait_gather(b):
        pltpu.make_async_copy(hidden_hbm.at[dst[b]], rows[b], sem_g[b]).wait()

    def issue_scatter(b):
        pltpu.async_copy(rows[b], acc.at[src[b]], sem_s[b], add=True)

    def wait_scatter(b):
        pltpu.make_async_copy(rows[b], acc.at[src[b]], sem_s[b]).wait()

    def scale(b):
        def _blk(i, _):
            pv = pr[b][pl.ds(i * 16, 16)]
            for r in range(16):
                row = i * 16 + r
                p = pv[r]
                for g in range(H // 16):
                    sl = pl.ds(g * 16, 16)
                    rows[b][row, sl] = rows[b][row, sl] * p
            return 0
        lax.fori_loop(0, C // 16, _blk, 0)

    def _zero_blk(r, _):
        for g in range(H // 16):
            rows0[r, pl.ds(g * 16, 16)] = jnp.zeros((16,), jnp.float32)
        return 0
    lax.fori_loop(0, C, _zero_blk, 0)
    base_row = s * RPT
    for j in range(RPT // C):
        pltpu.sync_copy(rows0, acc.at[pl.ds(base_row + j * C, C)])
    pltpu.sync_copy(rows0.at[pl.ds(0, RPT % C)],
                    acc.at[pl.ds(base_row + (RPT // C) * C, RPT % C)])
    plsc.subcore_barrier()

    issue_packed(0, 0)
    issue_packed(1, 1)
    issue_packed(2, 2)
    for k in (0, 1):
        wait_packed(k)
        unpack(k)
        issue_gather(k)
        issue_prob(k, k)
    for k in (0, 1):
        b2, b3 = (k + 2) % NBUF, (k + 3) % NBUF
        wait_packed(b2)
        unpack(b2)
        issue_gather(b2)
        issue_prob(b2, k + 2)
        issue_packed(b3, k + 3)
        wait_gather(k)
        wait_prob(k)
        scale(k)
        issue_scatter(k)

    def _group(k4, _):
        for bb in range(NBUF):
            k = 2 + k4 * NBUF + bb
            b = (2 + bb) % NBUF
            b2 = (b + 2) % NBUF
            b3 = (b + 3) % NBUF
            wait_scatter(b2)
            wait_packed(b2)
            unpack(b2)
            issue_gather(b2)
            issue_prob(b2, k + 2)
            issue_packed(b3, k + 3)
            wait_gather(b)
            wait_prob(b)
            scale(b)
            issue_scatter(b)
        return 0
    lax.fori_loop(0, (NCHUNK - 5) // NBUF, _group, 0)

    wait_scatter(0)
    wait_packed(0)
    unpack(0)
    issue_gather(0)
    issue_prob(0, NCHUNK - 1)
    wait_gather(2)
    wait_prob(2)
    scale(2)
    issue_scatter(2)
    wait_scatter(1)
    wait_gather(3)
    wait_prob(3)
    scale(3)
    issue_scatter(3)
    wait_scatter(2)
    wait_gather(0)
    wait_prob(0)
    scale(0)
    issue_scatter(0)
    wait_scatter(3)
    wait_scatter(0)
    plsc.subcore_barrier()

    for j in range(RPT // C):
        r0 = base_row + j * C
        pltpu.sync_copy(acc.at[pl.ds(r0, C)], rows0)
        pltpu.sync_copy(rows0, out_hbm.at[c, pl.ds(r0, C)])
    r0 = base_row + (RPT // C) * C
    rem = RPT % C
    pltpu.sync_copy(acc.at[pl.ds(r0, rem)], rows0.at[pl.ds(0, rem)])
    pltpu.sync_copy(rows0.at[pl.ds(0, rem)], out_hbm.at[c, pl.ds(r0, rem)])


_sc_call = functools.partial(
    pl.kernel,
    out_type=jax.ShapeDtypeStruct((NC, N_PAD, H), jnp.float32),
    mesh=plsc.VectorSubcoreMesh(core_axis_name="c", subcore_axis_name="s"),
    scratch_types=(
        [pltpu.VMEM((C,), jnp.int32) for _ in range(NBUF)]
        + [pltpu.VMEM((C,), jnp.int32) for _ in range(NBUF)]
        + [pltpu.VMEM((C,), jnp.int32) for _ in range(NBUF)]
        + [pltpu.VMEM((C,), jnp.float32) for _ in range(NBUF)]
        + [pltpu.VMEM((C, H), jnp.float32) for _ in range(NBUF)]
        + [pltpu.VMEM_SHARED((N_PAD, H), jnp.float32)]
        + [[pltpu.SemaphoreType.DMA] * NBUF for _ in range(4)]
    ),
)(_sc_aggregate)


def _tc_project(partials_ref, hs_ref, wrow_ref, wptr_ref, bias_ref, out_ref):
    agg = partials_ref[0, pl.ds(0, N)] + partials_ref[1, pl.ds(0, N)]
    enc_ptr = lax.dot_general(agg, wptr_ref[...], (((1,), (1,)), ((), ())),
                              preferred_element_type=jnp.float32)
    out_ref[...] = enc_ptr + hs_ref[...] * wrow_ref[...] + bias_ref[...]


def kernel(hint_scalar, hint_pointer, current_hidden, edge_index, step,
           W_scalar, b_scalar, W_ptr, b_ptr):
    src = edge_index[0]
    dst = edge_index[1]
    packed = src * 16384 + dst
    prob = jnp.take(hint_pointer, step, axis=1)
    hs = jnp.take(hint_scalar, step, axis=1)[:, None]

    partials = _sc_call(packed, prob, current_hidden)

    wrow = W_scalar.reshape(1, H)
    bias = (b_scalar + b_ptr).reshape(1, H)
    out = pl.pallas_call(
        _tc_project,
        out_shape=jax.ShapeDtypeStruct((N, H), jnp.float32),
    )(partials, hs, wrow, W_ptr, bias)
    return out
